# baseline (speedup 1.0000x reference)
import jax
import jax.numpy as jnp
from jax import lax
from jax.experimental import pallas as pl
from jax.experimental.pallas import tpu as pltpu
from jax.experimental.pallas import tpu_sc as plsc

X, Y, Z = 192, 160, 192
YZ = Y * Z
XYZ = X * Y * Z
N = 300
NUM_TILES = 16
PER_TILE = 32
NPAD = NUM_TILES * PER_TILE
L = 16


def _tre_body(vf_hbm, fx_hbm, fy_hbm, fz_hbm, mx_hbm, my_hbm, mz_hbm, sp_hbm,
              out_hbm,
              fx_v, fy_v, fz_v, mx_v, my_v, mz_v, sp_v, idx_v, disp_v,
              part_v, all_v, out_v, shared, sem):
    s = lax.axis_index("s")
    base = s * PER_TILE

    pltpu.sync_copy(fx_hbm.at[pl.ds(base, PER_TILE)], fx_v)
    pltpu.sync_copy(fy_hbm.at[pl.ds(base, PER_TILE)], fy_v)
    pltpu.sync_copy(fz_hbm.at[pl.ds(base, PER_TILE)], fz_v)
    pltpu.sync_copy(mx_hbm.at[pl.ds(base, PER_TILE)], mx_v)
    pltpu.sync_copy(my_hbm.at[pl.ds(base, PER_TILE)], my_v)
    pltpu.sync_copy(mz_hbm.at[pl.ds(base, PER_TILE)], mz_v)
    pltpu.sync_copy(sp_hbm, sp_v)

    for j in range(PER_TILE // L):
        fxj = fx_v[pl.ds(j * L, L)]
        fyj = fy_v[pl.ds(j * L, L)]
        fzj = fz_v[pl.ds(j * L, L)]
        flat = fxj * YZ + fyj * Z + fzj
        for c in range(3):
            idx_v[pl.ds(c * PER_TILE + j * L, L)] = flat + c * XYZ

    pltpu.async_copy(vf_hbm.at[idx_v], disp_v, sem).wait()

    sx = sp_v[pl.ds(0, L)]
    sy = sp_v[pl.ds(L, L)]
    sz = sp_v[pl.ds(2 * L, L)]
    acc = jnp.zeros((L,), jnp.float32)
    for j in range(PER_TILE // L):
        fxj = fx_v[pl.ds(j * L, L)].astype(jnp.float32)
        fyj = fy_v[pl.ds(j * L, L)].astype(jnp.float32)
        fzj = fz_v[pl.ds(j * L, L)].astype(jnp.float32)
        dx = (fxj + disp_v[pl.ds(0 * PER_TILE + j * L, L)] - mx_v[pl.ds(j * L, L)]) * sx
        dy = (fyj + disp_v[pl.ds(1 * PER_TILE + j * L, L)] - my_v[pl.ds(j * L, L)]) * sy
        dz = (fzj + disp_v[pl.ds(2 * PER_TILE + j * L, L)] - mz_v[pl.ds(j * L, L)]) * sz
        d2 = dx * dx + dy * dy + dz * dz
        n_global = base + j * L + lax.iota(jnp.int32, L)
        acc = acc + jnp.where(n_global < N, d2, 0.0)

    part_v[...] = acc
    pltpu.sync_copy(part_v, shared.at[pl.ds(s * L, L)])
    plsc.subcore_barrier()

    @pl.when(s == 0)
    def _():
        pltpu.sync_copy(shared, all_v)
        tot = jnp.zeros((L,), jnp.float32)
        for r in range(NUM_TILES):
            tot = tot + all_v[pl.ds(r * L, L)]
        total = tot[0]
        for i in range(1, L):
            total = total + tot[i]
        out_v[...] = jnp.full((L,), total * (1.0 / N), jnp.float32)
        pltpu.sync_copy(out_v, out_hbm)


@jax.jit
def _tre(vf_flat, fx, fy, fz, mx, my, mz, spb):
    mesh = plsc.VectorSubcoreMesh(
        core_axis_name="c", subcore_axis_name="s", num_cores=1)
    run = pl.kernel(
        _tre_body,
        out_type=jax.ShapeDtypeStruct((L,), jnp.float32),
        mesh=mesh,
        scratch_types=[
            pltpu.VMEM((PER_TILE,), jnp.int32),
            pltpu.VMEM((PER_TILE,), jnp.int32),
            pltpu.VMEM((PER_TILE,), jnp.int32),
            pltpu.VMEM((PER_TILE,), jnp.float32),
            pltpu.VMEM((PER_TILE,), jnp.float32),
            pltpu.VMEM((PER_TILE,), jnp.float32),
            pltpu.VMEM((3 * L,), jnp.float32),
            pltpu.VMEM((3 * PER_TILE,), jnp.int32),
            pltpu.VMEM((3 * PER_TILE,), jnp.float32),
            pltpu.VMEM((L,), jnp.float32),
            pltpu.VMEM((NUM_TILES * L,), jnp.float32),
            pltpu.VMEM((L,), jnp.float32),
            pltpu.VMEM_SHARED((NUM_TILES * L,), jnp.float32),
            pltpu.SemaphoreType.DMA,
        ],
    )
    return run(vf_flat, fx, fy, fz, mx, my, mz, spb)


def kernel(vector_field, moving_landmarks, fixed_landmarks, image_spacing):
    vf_flat = vector_field.reshape(3 * XYZ)
    fl = fixed_landmarks[0].astype(jnp.int32)
    ml = moving_landmarks[0]
    pad = NPAD - N
    fx = jnp.pad(fl[:, 0], (0, pad))
    fy = jnp.pad(fl[:, 1], (0, pad))
    fz = jnp.pad(fl[:, 2], (0, pad))
    mx = jnp.pad(ml[:, 0], (0, pad))
    my = jnp.pad(ml[:, 1], (0, pad))
    mz = jnp.pad(ml[:, 2], (0, pad))
    spb = jnp.repeat(image_spacing.astype(jnp.float32), L)
    out = _tre(vf_flat, fx, fy, fz, mx, my, mz, spb)
    return out[0]


# speedup vs baseline: 3.5853x; 3.5853x over previous
import jax
import jax.numpy as jnp
from jax import lax
from jax.experimental import pallas as pl
from jax.experimental.pallas import tpu as pltpu
from jax.experimental.pallas import tpu_sc as plsc

X, Y, Z = 192, 160, 192
N = 300
NUM_TILES = 16
PER_TILE = 32
NPAD = NUM_TILES * PER_TILE
L = 16
G = 3 * X * (Y // 8)
CHUNKS = 3 * PER_TILE


def _tre_body(f3_hbm, fx_hbm, fy_hbm, fz_hbm, mx_hbm, my_hbm, mz_hbm, sp_hbm,
              out_hbm,
              fx_v, fy_v, fz_v, mx_v, my_v, mz_v, sp_v, buf_v,
              part_v, all_v, out_v, shared, sem):
    s = lax.axis_index("s")
    base = s * PER_TILE

    pltpu.sync_copy(fx_hbm.at[pl.ds(base, PER_TILE)], fx_v)
    pltpu.sync_copy(fy_hbm.at[pl.ds(base, PER_TILE)], fy_v)
    pltpu.sync_copy(fz_hbm.at[pl.ds(base, PER_TILE)], fz_v)
    pltpu.sync_copy(mx_hbm.at[pl.ds(base, PER_TILE)], mx_v)
    pltpu.sync_copy(my_hbm.at[pl.ds(base, PER_TILE)], my_v)
    pltpu.sync_copy(mz_hbm.at[pl.ds(base, PER_TILE)], mz_v)
    pltpu.sync_copy(sp_hbm, sp_v)

    copies = []
    for j in range(PER_TILE // L):
        fxj = fx_v[pl.ds(j * L, L)]
        fyj = fy_v[pl.ds(j * L, L)]
        fzj = fz_v[pl.ds(j * L, L)]
        g0 = fxj * (Y // 8) + jnp.right_shift(fyj, 3)
        iy = jnp.bitwise_and(fyj, 7)
        zc = jnp.right_shift(fzj, 7)
        for i in range(L):
            n = j * L + i
            g = g0[i]
            iyi = iy[i]
            zoff = zc[i] * 128
            for c in range(3):
                row = c * PER_TILE + n
                copies.append(pltpu.async_copy(
                    f3_hbm.at[g + c * (X * Y // 8), iyi,
                              pl.ds(zoff, 128)],
                    buf_v.at[row], sem))
    for cp in copies:
        cp.wait()

    sx = sp_v[pl.ds(0, L)]
    sy = sp_v[pl.ds(L, L)]
    sz = sp_v[pl.ds(2 * L, L)]
    acc = jnp.zeros((L,), jnp.float32)
    for j in range(PER_TILE // L):
        fzj = fz_v[pl.ds(j * L, L)]
        lanes = jnp.bitwise_and(fzj, 127)
        rows0 = j * L + lax.iota(jnp.int32, L)
        fxj = fx_v[pl.ds(j * L, L)].astype(jnp.float32)
        fyj = fy_v[pl.ds(j * L, L)].astype(jnp.float32)
        fzf = fzj.astype(jnp.float32)
        dispx = plsc.load_gather(buf_v, [rows0, lanes])
        dispy = plsc.load_gather(buf_v, [rows0 + PER_TILE, lanes])
        dispz = plsc.load_gather(buf_v, [rows0 + 2 * PER_TILE, lanes])
        dx = (fxj + dispx - mx_v[pl.ds(j * L, L)]) * sx
        dy = (fyj + dispy - my_v[pl.ds(j * L, L)]) * sy
        dz = (fzf + dispz - mz_v[pl.ds(j * L, L)]) * sz
        d2 = dx * dx + dy * dy + dz * dz
        n_global = base + j * L + lax.iota(jnp.int32, L)
        acc = acc + jnp.where(n_global < N, d2, 0.0)

    part_v[...] = acc
    pltpu.sync_copy(part_v, shared.at[pl.ds(s * L, L)])
    plsc.subcore_barrier()

    @pl.when(s == 0)
    def _():
        pltpu.sync_copy(shared, all_v)
        tot = jnp.zeros((L,), jnp.float32)
        for r in range(NUM_TILES):
            tot = tot + all_v[pl.ds(r * L, L)]
        total = tot[0]
        for i in range(1, L):
            total = total + tot[i]
        out_v[...] = jnp.full((L,), total * (1.0 / N), jnp.float32)
        pltpu.sync_copy(out_v, out_hbm)


@jax.jit
def _tre(f3, fx, fy, fz, mx, my, mz, spb):
    mesh = plsc.VectorSubcoreMesh(
        core_axis_name="c", subcore_axis_name="s", num_cores=1)
    run = pl.kernel(
        _tre_body,
        out_type=jax.ShapeDtypeStruct((L,), jnp.float32),
        mesh=mesh,
        scratch_types=[
            pltpu.VMEM((PER_TILE,), jnp.int32),
            pltpu.VMEM((PER_TILE,), jnp.int32),
            pltpu.VMEM((PER_TILE,), jnp.int32),
            pltpu.VMEM((PER_TILE,), jnp.float32),
            pltpu.VMEM((PER_TILE,), jnp.float32),
            pltpu.VMEM((PER_TILE,), jnp.float32),
            pltpu.VMEM((3 * L,), jnp.float32),
            pltpu.VMEM((CHUNKS, 128), jnp.float32),
            pltpu.VMEM((L,), jnp.float32),
            pltpu.VMEM((NUM_TILES * L,), jnp.float32),
            pltpu.VMEM((L,), jnp.float32),
            pltpu.VMEM_SHARED((NUM_TILES * L,), jnp.float32),
            pltpu.SemaphoreType.DMA,
        ],
        compiler_params=pltpu.CompilerParams(
            use_tc_tiling_on_sc=True, needs_layout_passes=False),
    )
    return run(f3, fx, fy, fz, mx, my, mz, spb)


def kernel(vector_field, moving_landmarks, fixed_landmarks, image_spacing):
    f3 = vector_field.reshape(G, 8, Z)
    fl = fixed_landmarks[0].astype(jnp.int32)
    ml = moving_landmarks[0]
    pad = NPAD - N
    fx = jnp.pad(fl[:, 0], (0, pad))
    fy = jnp.pad(fl[:, 1], (0, pad))
    fz = jnp.pad(fl[:, 2], (0, pad))
    mx = jnp.pad(ml[:, 0], (0, pad))
    my = jnp.pad(ml[:, 1], (0, pad))
    mz = jnp.pad(ml[:, 2], (0, pad))
    spb = jnp.repeat(image_spacing.astype(jnp.float32), L)
    out = _tre(f3, fx, fy, fz, mx, my, mz, spb)
    return out[0]


# speedup vs baseline: 4.2230x; 1.1778x over previous
import jax
import jax.numpy as jnp
from jax import lax
from jax.experimental import pallas as pl
from jax.experimental.pallas import tpu as pltpu
from jax.experimental.pallas import tpu_sc as plsc

X, Y, Z = 192, 160, 192
N = 300
NUM_TILES = 16
PER_TILE = 32
NPAD = NUM_TILES * PER_TILE
L = 16
R = 3 * X * Y
CHUNKS = 3 * PER_TILE

OFF_FX, OFF_FY, OFF_FZ = 0, 32, 64
OFF_MX, OFF_MY, OFF_MZ = 96, 128, 160
OFF_SP = 192
PACK_W = 256


def _tre_body(f2_hbm, pk_hbm, out_hbm,
              pk_v, buf_v, part_v, all_v, out_v, shared, sem):
    s = lax.axis_index("s")
    base = s * PER_TILE

    pltpu.sync_copy(pk_hbm.at[pl.ds(s * PACK_W, PACK_W)], pk_v)

    copies = []
    for j in range(PER_TILE // L):
        fxj = pk_v[pl.ds(OFF_FX + j * L, L)]
        fyj = pk_v[pl.ds(OFF_FY + j * L, L)]
        fzj = pk_v[pl.ds(OFF_FZ + j * L, L)]
        g0 = fxj * (Y // 8) + jnp.right_shift(fyj, 3)
        iyj = jnp.bitwise_and(fyj, 7)
        zc = jnp.right_shift(fzj, 7)
        for i in range(L):
            n = j * L + i
            g = g0[i]
            iyi = iyj[i]
            zoff = zc[i] * 128
            for c in range(3):
                copies.append(pltpu.async_copy(
                    f2_hbm.at[g + c * (X * Y // 8), iyi, pl.ds(zoff, 128)],
                    buf_v.at[c * PER_TILE + n], sem))
    for cp in copies:
        cp.wait()

    sx = plsc.bitcast(pk_v[pl.ds(OFF_SP, L)], jnp.float32)
    sy = plsc.bitcast(pk_v[pl.ds(OFF_SP + L, L)], jnp.float32)
    sz = plsc.bitcast(pk_v[pl.ds(OFF_SP + 2 * L, L)], jnp.float32)
    acc = jnp.zeros((L,), jnp.float32)
    for j in range(PER_TILE // L):
        fxj = pk_v[pl.ds(OFF_FX + j * L, L)]
        fyj = pk_v[pl.ds(OFF_FY + j * L, L)]
        fzj = pk_v[pl.ds(OFF_FZ + j * L, L)]
        mxj = plsc.bitcast(pk_v[pl.ds(OFF_MX - OFF_FX + OFF_FX + j * L, L)], jnp.float32)
        myj = plsc.bitcast(pk_v[pl.ds(OFF_MY - OFF_FY + OFF_FY + j * L, L)], jnp.float32)
        mzj = plsc.bitcast(pk_v[pl.ds(OFF_MZ - OFF_FZ + OFF_FZ + j * L, L)], jnp.float32)
        lanes = jnp.bitwise_and(fzj, 127)
        rows0 = j * L + lax.iota(jnp.int32, L)
        dispx = plsc.load_gather(buf_v, [rows0, lanes])
        dispy = plsc.load_gather(buf_v, [rows0 + PER_TILE, lanes])
        dispz = plsc.load_gather(buf_v, [rows0 + 2 * PER_TILE, lanes])
        dx = (fxj.astype(jnp.float32) + dispx - mxj) * sx
        dy = (fyj.astype(jnp.float32) + dispy - myj) * sy
        dz = (fzj.astype(jnp.float32) + dispz - mzj) * sz
        d2 = dx * dx + dy * dy + dz * dz
        n_global = base + j * L + lax.iota(jnp.int32, L)
        acc = acc + jnp.where(n_global < N, d2, 0.0)

    part_v[...] = acc
    pltpu.sync_copy(part_v, shared.at[pl.ds(s * L, L)])
    plsc.subcore_barrier()

    @pl.when(s == 0)
    def _():
        pltpu.sync_copy(shared, all_v)
        tot = jnp.zeros((L,), jnp.float32)
        for r in range(NUM_TILES):
            tot = tot + all_v[pl.ds(r * L, L)]
        total = tot[0]
        for i in range(1, L):
            total = total + tot[i]
        out_v[...] = jnp.full((L,), total * (1.0 / N), jnp.float32)
        pltpu.sync_copy(out_v, out_hbm)


@jax.jit
def _tre(f2, pk):
    mesh = plsc.VectorSubcoreMesh(
        core_axis_name="c", subcore_axis_name="s", num_cores=1)
    run = pl.kernel(
        _tre_body,
        out_type=jax.ShapeDtypeStruct((L,), jnp.float32),
        mesh=mesh,
        scratch_types=[
            pltpu.VMEM((PACK_W,), jnp.int32),
            pltpu.VMEM((CHUNKS, 128), jnp.float32),
            pltpu.VMEM((L,), jnp.float32),
            pltpu.VMEM((NUM_TILES * L,), jnp.float32),
            pltpu.VMEM((L,), jnp.float32),
            pltpu.VMEM_SHARED((NUM_TILES * L,), jnp.float32),
            pltpu.SemaphoreType.DMA,
        ],
        compiler_params=pltpu.CompilerParams(
            use_tc_tiling_on_sc=True, needs_layout_passes=False),
    )
    return run(f2, pk)


def kernel(vector_field, moving_landmarks, fixed_landmarks, image_spacing):
    f2 = vector_field.reshape(R // 8, 8, Z)
    fl = fixed_landmarks[0].astype(jnp.int32)
    mlb = jax.lax.bitcast_convert_type(moving_landmarks[0], jnp.int32)
    pad = NPAD - N
    flp = jnp.pad(fl, ((0, pad), (0, 0)))
    flp = flp.reshape(NUM_TILES, PER_TILE, 3).transpose(0, 2, 1)
    flp = flp.reshape(NUM_TILES, 3 * PER_TILE)
    mlp = jnp.pad(mlb, ((0, pad), (0, 0)))
    mlp = mlp.reshape(NUM_TILES, PER_TILE, 3).transpose(0, 2, 1)
    mlp = mlp.reshape(NUM_TILES, 3 * PER_TILE)
    spb = jnp.repeat(
        jax.lax.bitcast_convert_type(image_spacing.astype(jnp.float32),
                                     jnp.int32), L)
    spt = jnp.broadcast_to(spb, (NUM_TILES, 3 * L))
    pad16 = jnp.zeros((NUM_TILES, PACK_W - OFF_SP - 3 * L), jnp.int32)
    pk = jnp.concatenate([flp, mlp, spt, pad16], axis=1)
    pk = pk.reshape(NUM_TILES * PACK_W)
    out = _tre(f2, pk)
    return out[0]
